# Initial kernel scaffold; baseline (speedup 1.0000x reference)
import jax
import jax.numpy as jnp
from jax.experimental import pallas as pl


def kernel(target_ixs, context_ixs, negative_sample_ixs, word_embeds, context_embeds, rho_state0):
    raise NotImplementedError("write your pallas kernel here")


# speedup vs baseline: 5.3728x; 5.3728x over previous
import math

import jax
import jax.numpy as jnp
from jax import lax
from jax.experimental import pallas as pl
from jax.experimental.pallas import tpu as pltpu
from jax.experimental.pallas import tpu_sc as plsc

N_VOCAB = 100000
N_DIM = 64
SIGMA = 1.0
B = 16384
CS = 20
NS = 20
NR = NS + 1

NC = 2
NSUB = 16
NW = NC * NSUB
BPW = B // NW
NB = 32
NBLK = BPW // NB
PW = NR * 16


def _logits_sc_body(wixs, cixs, wtab, ctab, out, widx_v, cidx_v,
                    wrows_v, crows_v, out_v, sem_w, sem_c):
    wid = lax.axis_index("s") * NC + lax.axis_index("c")

    def block(blk, carry):
        base = wid * BPW + blk * NB
        pltpu.sync_copy(wixs.at[pl.ds(base * NR, NB * NR)], widx_v)
        pltpu.sync_copy(cixs.at[pl.ds(base * CS, NB * CS)], cidx_v)
        cw = pltpu.async_copy(wtab.at[widx_v], wrows_v, sem_w)
        cc = pltpu.async_copy(ctab.at[cidx_v], crows_v, sem_c)
        cw.wait()
        cc.wait()

        def elem(i, c2):
            cacc = [crows_v[i * CS, pl.ds(16 * k, 16)] for k in range(4)]
            for j in range(1, CS):
                for k in range(4):
                    cacc[k] = cacc[k] + crows_v[i * CS + j, pl.ds(16 * k, 16)]
            for r in range(NR):
                p = wrows_v[i * NR + r, pl.ds(0, 16)] * cacc[0]
                for k in range(1, 4):
                    p = p + wrows_v[i * NR + r, pl.ds(16 * k, 16)] * cacc[k]
                out_v[i, pl.ds(16 * r, 16)] = p
            return c2

        lax.fori_loop(0, NB, elem, 0)
        pltpu.sync_copy(out_v, out.at[pl.ds(base, NB)])
        return carry

    lax.fori_loop(0, NBLK, block, 0)


_logits_sc = pl.kernel(
    _logits_sc_body,
    mesh=plsc.VectorSubcoreMesh(core_axis_name="c", subcore_axis_name="s"),
    compiler_params=pltpu.CompilerParams(use_tc_tiling_on_sc=False),
    out_type=jax.ShapeDtypeStruct((B, PW), jnp.float32),
    scratch_types=[
        pltpu.VMEM((NB * NR,), jnp.int32),
        pltpu.VMEM((NB * CS,), jnp.int32),
        pltpu.VMEM((NB * NR, N_DIM), jnp.float32),
        pltpu.VMEM((NB * CS, N_DIM), jnp.float32),
        pltpu.VMEM((NB, PW), jnp.float32),
        pltpu.SemaphoreType.DMA,
        pltpu.SemaphoreType.DMA,
    ],
)


ROWS_BLK = 2000
GRID = N_VOCAB // ROWS_BLK
LG_BLK = 328
_PRIOR_CONST = float(N_VOCAB * N_DIM) * (
    -1.5 * math.log(2.0 * math.pi) - math.log(SIGMA) - math.log(SIGMA)
    - math.log(SIGMA / 100.0))
_INV2S2_3 = 0.5 * (100.0 / SIGMA) ** 2


def _loss_tc_body(w_ref, c_ref, r_ref, lg_ref, out_ref, acc_ref):
    step = pl.program_id(0)

    @pl.when(step == 0)
    def _init():
        acc_ref[...] = jnp.zeros((8, 128), jnp.float32)

    w = w_ref[...]
    c = c_ref[...]
    d = w - r_ref[...]
    part = (-0.5 / (SIGMA * SIGMA)) * (w * w + c * c) - _INV2S2_3 * (d * d)
    acc_ref[0:1, 0:N_DIM] += jnp.sum(part, axis=0, keepdims=True)

    x = lg_ref[...]
    gsel = (lax.broadcasted_iota(jnp.int32, (PW, NR), 0) // 16
            == lax.broadcasted_iota(jnp.int32, (PW, NR), 1)
            ).astype(jnp.float32)
    lg = lax.dot(x, gsel, precision=lax.Precision.HIGHEST)
    col = lax.broadcasted_iota(jnp.int32, (LG_BLK, NR), 1)
    y = jnp.where(col == 0, -lg, lg)
    sp = jnp.maximum(y, 0.0) + jnp.log1p(jnp.exp(-jnp.abs(y)))
    rowi = lax.broadcasted_iota(jnp.int32, (LG_BLK, NR), 0)
    nvalid = jnp.where(step == GRID - 1, B - (GRID - 1) * LG_BLK, LG_BLK)
    contrib = jnp.where(rowi < nvalid, -sp, 0.0)
    acc_ref[1:2, 0:NR] += jnp.sum(contrib, axis=0, keepdims=True)

    @pl.when(step == GRID - 1)
    def _fin():
        out_ref[0, 0] = jnp.sum(acc_ref[...]) + _PRIOR_CONST


def _loss_tc(word_embeds, context_embeds, rho_state0, partials):
    return pl.pallas_call(
        _loss_tc_body,
        grid=(GRID,),
        in_specs=[
            pl.BlockSpec((ROWS_BLK, N_DIM), lambda i: (i, 0)),
            pl.BlockSpec((ROWS_BLK, N_DIM), lambda i: (i, 0)),
            pl.BlockSpec((ROWS_BLK, N_DIM), lambda i: (i, 0)),
            pl.BlockSpec((LG_BLK, PW), lambda i: (i, 0)),
        ],
        out_specs=pl.BlockSpec(memory_space=pltpu.SMEM),
        out_shape=jax.ShapeDtypeStruct((1, 1), jnp.float32),
        scratch_shapes=[pltpu.VMEM((8, 128), jnp.float32)],
    )(word_embeds, context_embeds, rho_state0, partials)


def kernel(target_ixs, context_ixs, negative_sample_ixs, word_embeds,
           context_embeds, rho_state0):
    wixs = jnp.concatenate(
        [target_ixs[:, None], negative_sample_ixs], axis=1).reshape(-1)
    cixs = context_ixs.reshape(-1)
    partials = _logits_sc(wixs, cixs, word_embeds, context_embeds)
    loss = _loss_tc(word_embeds, context_embeds, rho_state0, partials)
    return loss[0, 0]
